# baseline (speedup 1.0000x reference)
import numpy as np
import jax
import jax.numpy as jnp
from jax.experimental import pallas as pl
from jax.experimental.pallas import tpu as pltpu

ALPHA = 0.98
DELTA = 2.0
S = 0.025
A = 1.0 - S
EPS = 1e-6
SQRT_DELTA = DELTA ** 0.5

L = 256
M_TILE = 128
INB = 8
ONB = 4


def _coeff_matrix() -> np.ndarray:
    i = np.arange(L, dtype=np.float64)[:, None]
    k = np.arange(L, dtype=np.float64)[None, :]
    d = k - i
    C = np.where(d >= 0, S * np.power(A, np.maximum(d, 0.0)), 0.0)
    return C.astype(np.float32)


def _pow_row() -> np.ndarray:
    return (A ** (np.arange(L, dtype=np.float64) + 1.0)).astype(np.float32)[None, :]


def _make_body(n_tiles: int, t_len: int):
    def body(c_ref, p_ref, x_hbm, o_hbm, inb, outb, insem, outsem):
        C = c_ref[...]
        powrow = p_ref[...]

        def in_copy(i):
            s = jax.lax.rem(i, INB)
            return pltpu.make_async_copy(
                x_hbm.at[pl.ds(i * M_TILE, M_TILE), :], inb.at[s], insem.at[s])

        def out_copy(i):
            s = jax.lax.rem(i, ONB)
            return pltpu.make_async_copy(
                outb.at[s], o_hbm.at[pl.ds(i * M_TILE, M_TILE), :], outsem.at[s])

        for i in range(min(INB, n_tiles)):
            in_copy(i).start()

        def step(i, _):
            in_copy(i).wait()

            @pl.when(i >= ONB)
            def _():
                out_copy(i - ONB).wait()

            islot = jax.lax.rem(i, INB)
            oslot = jax.lax.rem(i, ONB)
            carry = inb[islot, :, 0:1]
            for j in range(t_len // L):
                xc = inb[islot, :, j * L:(j + 1) * L]
                local = jax.lax.dot_general(
                    xc, C, (((1,), (0,)), ((), ())),
                    preferred_element_type=jnp.float32)
                sm = local + carry * powrow
                carry = sm[:, L - 1:L]
                v = xc * jnp.exp2(-ALPHA * jnp.log2(sm + EPS)) + DELTA
                outb[oslot, :, j * L:(j + 1) * L] = v * jax.lax.rsqrt(v) - SQRT_DELTA

            out_copy(i).start()

            @pl.when(i + INB < n_tiles)
            def _():
                in_copy(i + INB).start()

            return 0

        jax.lax.fori_loop(0, n_tiles, step, 0, unroll=False)

        for i in range(max(n_tiles - ONB, 0), n_tiles):
            out_copy(i).wait()

    return body


def kernel(x):
    B, Cdim, T = x.shape
    M = B * Cdim
    n_tiles = M // M_TILE
    xf = x.reshape(M, T)
    Cm = jnp.asarray(_coeff_matrix())
    pr = jnp.asarray(_pow_row())
    out = pl.pallas_call(
        _make_body(n_tiles, T),
        out_shape=jax.ShapeDtypeStruct((M, T), jnp.float32),
        in_specs=[
            pl.BlockSpec(memory_space=pltpu.VMEM),
            pl.BlockSpec(memory_space=pltpu.VMEM),
            pl.BlockSpec(memory_space=pl.ANY),
        ],
        out_specs=pl.BlockSpec(memory_space=pl.ANY),
        scratch_shapes=[
            pltpu.VMEM((INB, M_TILE, T), jnp.float32),
            pltpu.VMEM((ONB, M_TILE, T), jnp.float32),
            pltpu.SemaphoreType.DMA((INB,)),
            pltpu.SemaphoreType.DMA((ONB,)),
        ],
        compiler_params=pltpu.CompilerParams(
            vmem_limit_bytes=56 * 1024 * 1024,
        ),
        name="pcen",
    )(Cm, pr, xf)
    return out.reshape(B, Cdim, T)


# speedup vs baseline: 1.0978x; 1.0978x over previous
import numpy as np
import jax
import jax.numpy as jnp
from jax.experimental import pallas as pl
from jax.experimental.pallas import tpu as pltpu

ALPHA = 0.98
DELTA = 2.0
S = 0.025
A = 1.0 - S
EPS = 1e-6
SQRT_DELTA = DELTA ** 0.5

L = 256
M_TILE = 256
INB = 6
ONB = 4


def _coeff_matrix() -> np.ndarray:
    i = np.arange(L, dtype=np.float64)[:, None]
    k = np.arange(L, dtype=np.float64)[None, :]
    d = k - i
    C = np.where(d >= 0, S * np.power(A, np.maximum(d, 0.0)), 0.0)
    return C.astype(np.float32)


def _pow_row() -> np.ndarray:
    return (A ** (np.arange(L, dtype=np.float64) + 1.0)).astype(np.float32)[None, :]


def _make_body(n_tiles: int, t_len: int):
    def body(c_ref, p_ref, x_hbm, o_hbm, inb, outb, insem, outsem):
        C = c_ref[...]
        powrow = p_ref[...]

        def in_copy(i):
            s = jax.lax.rem(i, INB)
            return pltpu.make_async_copy(
                x_hbm.at[pl.ds(i * M_TILE, M_TILE), :], inb.at[s], insem.at[s])

        def out_copy(i):
            s = jax.lax.rem(i, ONB)
            return pltpu.make_async_copy(
                outb.at[s], o_hbm.at[pl.ds(i * M_TILE, M_TILE), :], outsem.at[s])

        for i in range(min(INB, n_tiles)):
            in_copy(i).start()

        def step(i, _):
            in_copy(i).wait()

            @pl.when(i >= ONB)
            def _():
                out_copy(i - ONB).wait()

            islot = jax.lax.rem(i, INB)
            oslot = jax.lax.rem(i, ONB)
            carry = inb[islot, :, 0:1]
            for j in range(t_len // L):
                xc = inb[islot, :, j * L:(j + 1) * L]
                local = jax.lax.dot_general(
                    xc, C, (((1,), (0,)), ((), ())),
                    preferred_element_type=jnp.float32)
                sm = local + carry * powrow
                carry = sm[:, L - 1:L]
                v = xc * jnp.exp2(-ALPHA * jnp.log2(sm + EPS)) + DELTA
                outb[oslot, :, j * L:(j + 1) * L] = v * jax.lax.rsqrt(v) - SQRT_DELTA

            out_copy(i).start()

            @pl.when(i + INB < n_tiles)
            def _():
                in_copy(i + INB).start()

            return 0

        jax.lax.fori_loop(0, n_tiles, step, 0, unroll=False)

        for i in range(max(n_tiles - ONB, 0), n_tiles):
            out_copy(i).wait()

    return body


def kernel(x):
    B, Cdim, T = x.shape
    M = B * Cdim
    n_tiles = M // M_TILE
    xf = x.reshape(M, T)
    Cm = jnp.asarray(_coeff_matrix())
    pr = jnp.asarray(_pow_row())
    out = pl.pallas_call(
        _make_body(n_tiles, T),
        out_shape=jax.ShapeDtypeStruct((M, T), jnp.float32),
        in_specs=[
            pl.BlockSpec(memory_space=pltpu.VMEM),
            pl.BlockSpec(memory_space=pltpu.VMEM),
            pl.BlockSpec(memory_space=pl.ANY),
        ],
        out_specs=pl.BlockSpec(memory_space=pl.ANY),
        scratch_shapes=[
            pltpu.VMEM((INB, M_TILE, T), jnp.float32),
            pltpu.VMEM((ONB, M_TILE, T), jnp.float32),
            pltpu.SemaphoreType.DMA((INB,)),
            pltpu.SemaphoreType.DMA((ONB,)),
        ],
        compiler_params=pltpu.CompilerParams(
            vmem_limit_bytes=56 * 1024 * 1024,
        ),
        name="pcen",
    )(Cm, pr, xf)
    return out.reshape(B, Cdim, T)
